# baseline (speedup 1.0000x reference)
import functools
import jax
import jax.numpy as jnp
from jax import lax
from jax.experimental import pallas as pl
from jax.experimental.pallas import tpu as pltpu
from jax.experimental.pallas import tpu_sc as plsc

D = 128
N_NODES = 10000
N_EDGES = 320000
E2 = N_EDGES // 2
N_LAYERS = 5
EPS = 1e-5


def _encode_body(x_ref, at_ref, wt_ref, h_ref, hw_ref):
    xb = x_ref[...]
    acc = jnp.zeros((x_ref.shape[0], D), jnp.float32)
    for f in range(at_ref.shape[0]):
        oh = (xb[:, f][:, None] == lax.broadcasted_iota(
            jnp.int32, (x_ref.shape[0], at_ref.shape[1]), 1)).astype(jnp.float32)
        acc = acc + jnp.dot(oh, at_ref[f], preferred_element_type=jnp.float32)
    h_ref[...] = acc
    hw_ref[...] = jnp.dot(acc, wt_ref[...], preferred_element_type=jnp.float32)


def tc_encode(x, atom_tables, w_top):
    Bn = 2000
    grid = (N_NODES // Bn,)
    return pl.pallas_call(
        _encode_body,
        grid=grid,
        in_specs=[
            pl.BlockSpec((Bn, x.shape[1]), lambda i: (i, 0)),
            pl.BlockSpec(atom_tables.shape, lambda i: (0, 0, 0)),
            pl.BlockSpec((D, D), lambda i: (0, 0)),
        ],
        out_specs=[
            pl.BlockSpec((Bn, D), lambda i: (i, 0)),
            pl.BlockSpec((Bn, D), lambda i: (i, 0)),
        ],
        out_shape=[
            jax.ShapeDtypeStruct((N_NODES, D), jnp.float32),
            jax.ShapeDtypeStruct((N_NODES, D), jnp.float32),
        ],
    )(x, atom_tables, w_top)


def _msg0_body(g_ref, ea_ref, bt_ref, wb_ref, o_ref):
    ea = ea_ref[...]
    acc = jnp.zeros((ea_ref.shape[0], D), jnp.float32)
    for f in range(bt_ref.shape[0]):
        oh = (ea[:, f][:, None] == lax.broadcasted_iota(
            jnp.int32, (ea_ref.shape[0], bt_ref.shape[1]), 1)).astype(jnp.float32)
        acc = acc + jnp.dot(oh, bt_ref[f], preferred_element_type=jnp.float32)
    o_ref[...] = g_ref[...] + jnp.dot(acc, wb_ref[...],
                                      preferred_element_type=jnp.float32)


def tc_msg0(g0, edge_attr, bond_tables, w_bot):
    Be = 4000
    grid = (N_EDGES // Be,)
    return pl.pallas_call(
        _msg0_body,
        grid=grid,
        in_specs=[
            pl.BlockSpec((Be, D), lambda i: (i, 0)),
            pl.BlockSpec((Be, edge_attr.shape[1]), lambda i: (i, 0)),
            pl.BlockSpec(bond_tables.shape, lambda i: (0, 0, 0)),
            pl.BlockSpec((D, D), lambda i: (0, 0)),
        ],
        out_specs=pl.BlockSpec((Be, D), lambda i: (i, 0)),
        out_shape=jax.ShapeDtypeStruct((N_EDGES, D), jnp.float32),
    )(g0, edge_attr, bond_tables, w_bot)


def _sum_parts_body(p_ref, o_ref):
    o_ref[...] = p_ref[0] + p_ref[1]


def tc_sum_parts(parts):
    Bn = 2000
    grid = (N_NODES // Bn,)
    return pl.pallas_call(
        _sum_parts_body,
        grid=grid,
        in_specs=[pl.BlockSpec((2, Bn, D), lambda i: (0, i, 0))],
        out_specs=pl.BlockSpec((Bn, D), lambda i: (i, 0)),
        out_shape=jax.ShapeDtypeStruct((N_NODES, D), jnp.float32),
    )(parts)


def _mm_stats_body(g_ref, m_ref, w_ref, y_ref, s_ref):
    g = g_ref[...]
    m = m_ref[...]
    up = jnp.concatenate([m[1:], m[:1]], axis=0)
    dn = jnp.concatenate([m[-1:], m[:-1]], axis=0)
    even = (lax.broadcasted_iota(jnp.int32, m.shape, 0) % 2) == 0
    u = g - jnp.where(even, up, dn)
    y = jnp.dot(u, w_ref[...], preferred_element_type=jnp.float32)
    y_ref[...] = y

    @pl.when(pl.program_id(0) == 0)
    def _():
        s_ref[...] = jnp.zeros_like(s_ref)

    s_ref[0, :] += jnp.sum(y, axis=0)
    s_ref[1, :] += jnp.sum(y * y, axis=0)


def tc_matmul_stats(g, msg, w_hidden):
    Be = 6400
    grid = (N_EDGES // Be,)
    return pl.pallas_call(
        _mm_stats_body,
        grid=grid,
        in_specs=[
            pl.BlockSpec((Be, D), lambda i: (i, 0)),
            pl.BlockSpec((Be, D), lambda i: (i, 0)),
            pl.BlockSpec((D, D), lambda i: (0, 0)),
        ],
        out_specs=[
            pl.BlockSpec((Be, D), lambda i: (i, 0)),
            pl.BlockSpec((8, D), lambda i: (0, 0)),
        ],
        out_shape=[
            jax.ShapeDtypeStruct((N_EDGES, D), jnp.float32),
            jax.ShapeDtypeStruct((8, D), jnp.float32),
        ],
    )(g, msg, w_hidden)


def _affine_relu_body(y_ref, a_ref, b_ref, o_ref):
    o_ref[...] = jnp.maximum(y_ref[...] * a_ref[...] + b_ref[...], 0.0)


def tc_affine_relu(y, a, b):
    Be = 8000
    grid = (N_EDGES // Be,)
    return pl.pallas_call(
        _affine_relu_body,
        grid=grid,
        in_specs=[
            pl.BlockSpec((Be, D), lambda i: (i, 0)),
            pl.BlockSpec((1, D), lambda i: (0, 0)),
            pl.BlockSpec((1, D), lambda i: (0, 0)),
        ],
        out_specs=pl.BlockSpec((Be, D), lambda i: (i, 0)),
        out_shape=jax.ShapeDtypeStruct((N_EDGES, D), jnp.float32),
    )(y, a, b)


def _final_body(h_ref, p_ref, wt_ref, wb_ref, b_ref, o_ref):
    nm = p_ref[0] + p_ref[1]
    r = (jnp.dot(h_ref[...], wt_ref[...], preferred_element_type=jnp.float32)
         + jnp.dot(nm, wb_ref[...], preferred_element_type=jnp.float32)
         + b_ref[...])
    o_ref[...] = jnp.maximum(r, 0.0)


def tc_final(h, parts, w_top, w_bot, bias):
    Bn = 2000
    grid = (N_NODES // Bn,)
    return pl.pallas_call(
        _final_body,
        grid=grid,
        in_specs=[
            pl.BlockSpec((Bn, D), lambda i: (i, 0)),
            pl.BlockSpec((2, Bn, D), lambda i: (0, i, 0)),
            pl.BlockSpec((D, D), lambda i: (0, 0)),
            pl.BlockSpec((D, D), lambda i: (0, 0)),
            pl.BlockSpec((1, D), lambda i: (0, 0)),
        ],
        out_specs=pl.BlockSpec((Bn, D), lambda i: (i, 0)),
        out_shape=jax.ShapeDtypeStruct((N_NODES, D), jnp.float32),
    )(h, parts, w_top, w_bot, bias)


NC = 2
NS = 16
NW = NC * NS
IDX_ROWS = N_EDGES // 128
CPW = 80
IDX_PAD = NW * CPW
NPAD = 10112

CPW_G = 88
IDX_PAD_G = NW * CPW_G
CH_G = CPW_G // 2
OPAD = IDX_PAD_G * 128


def sc_gather(table, idx_rows):
    mesh = plsc.VectorSubcoreMesh(core_axis_name="c", subcore_axis_name="s")

    @functools.partial(
        pl.kernel, mesh=mesh,
        out_type=jax.ShapeDtypeStruct((OPAD, D), jnp.float32),
        scratch_types=[
            pltpu.VMEM((CPW_G, 128), jnp.int32),
            pltpu.VMEM((3, 256, D), jnp.float32),
            pltpu.SemaphoreType.DMA((3,)),
            pltpu.SemaphoreType.DMA((3,)),
        ],
    )
    def k(table_hbm, idx_hbm, out_hbm, idx_v, rows_v, gsem, ssem):
        c = lax.axis_index("c")
        s = lax.axis_index("s")
        wid = s * NC + c
        lo = wid * CPW_G
        pltpu.sync_copy(idx_hbm.at[pl.ds(lo, CPW_G)], idx_v)

        def gath(q, b):
            pltpu.async_copy(table_hbm.at[idx_v.at[2 * q]],
                             rows_v.at[b, pl.ds(0, 128)], gsem.at[b])
            pltpu.async_copy(table_hbm.at[idx_v.at[2 * q + 1]],
                             rows_v.at[b, pl.ds(128, 128)], gsem.at[b])

        def put(q, b):
            pltpu.make_async_copy(table_hbm.at[idx_v.at[2 * q]],
                                  rows_v.at[b, pl.ds(0, 128)], gsem.at[b]).wait()
            pltpu.make_async_copy(table_hbm.at[idx_v.at[2 * q + 1]],
                                  rows_v.at[b, pl.ds(128, 128)], gsem.at[b]).wait()
            pltpu.async_copy(rows_v.at[b],
                             out_hbm.at[pl.ds((lo + 2 * q) * 128, 256)],
                             ssem.at[b])

        def drain(q, b):
            @pl.when(q >= 0)
            def _():
                pltpu.make_async_copy(rows_v.at[b],
                                      out_hbm.at[pl.ds((lo + 2 * q) * 128, 256)],
                                      ssem.at[b]).wait()

        gath(0, 0)
        gath(1, 1)

        @pl.loop(0, (CH_G - 2) // 3)
        def _(i):
            for kk in range(3):
                q = i * 3 + kk
                b = kk % 3
                put(q, b)
                drain(q - 1, (kk + 2) % 3)

                @pl.when(q + 2 < CH_G)
                def _():
                    gath(q + 2, (kk + 2) % 3)

        put(CH_G - 2, (CH_G - 2) % 3)
        put(CH_G - 1, (CH_G - 1) % 3)
        drain(CH_G - 3, (CH_G - 3) % 3)
        drain(CH_G - 2, (CH_G - 2) % 3)
        drain(CH_G - 1, (CH_G - 1) % 3)

    return k(table, idx_rows)


def sc_scatter(msg, scat_idx, zeros):
    mesh = plsc.VectorSubcoreMesh(core_axis_name="c", subcore_axis_name="s")
    rows_per_core = IDX_ROWS // NC
    core_pad = NS * CPW
    stripe = NPAD // NS

    @functools.partial(
        pl.kernel, mesh=mesh,
        out_type=jax.ShapeDtypeStruct((NC, NPAD, D), jnp.float32),
        scratch_types=[
            pltpu.VMEM((CPW, 128), jnp.int32),
            pltpu.VMEM((2, 128, D), jnp.float32),
            pltpu.VMEM_SHARED((NPAD, D), jnp.float32),
            pltpu.SemaphoreType.DMA((2,)),
        ],
    )
    def k(msg_hbm, idx_hbm, z_hbm, out_hbm, idx_v, rows_v, acc_sh, lsem):
        c = lax.axis_index("c")
        s = lax.axis_index("s")
        pltpu.sync_copy(z_hbm.at[pl.ds(s * stripe, stripe)],
                        acc_sh.at[pl.ds(s * stripe, stripe)])
        base_l = s * CPW
        pltpu.sync_copy(idx_hbm.at[pl.ds(c * core_pad + base_l, CPW)], idx_v)
        plsc.subcore_barrier()

        def load(u, b):
            @pl.when((u >= 0) & (u < CPW) & (base_l + u < rows_per_core))
            def _():
                pltpu.async_copy(
                    msg_hbm.at[pl.ds((c * rows_per_core + base_l + u) * 128, 128)],
                    rows_v.at[b], lsem.at[b])

        def add(u, b):
            @pl.when(base_l + u < rows_per_core)
            def _():
                pltpu.make_async_copy(
                    msg_hbm.at[pl.ds((c * rows_per_core + base_l + u) * 128, 128)],
                    rows_v.at[b], lsem.at[b]).wait()
                pltpu.sync_copy(rows_v.at[b], acc_sh.at[idx_v.at[u]], add=True)

        load(0, 0)
        load(1, 1)

        @pl.loop(0, CPW // 2)
        def _(i):
            for kk in range(2):
                u = i * 2 + kk
                add(u, kk)
                load(u + 2, kk)

        plsc.subcore_barrier()
        pltpu.sync_copy(acc_sh.at[pl.ds(s * stripe, stripe)],
                        out_hbm.at[c, pl.ds(s * stripe, stripe)])

    return k(msg, scat_idx, zeros)


def kernel(x, edge_index, edge_attr, atom_tables, bond_tables,
           W_input, W_hidden, W_output, b_output, bn_gamma, bn_beta):
    x = x.astype(jnp.int32)
    edge_attr = edge_attr.astype(jnp.int32)
    src = edge_index[0].astype(jnp.int32)
    dst = edge_index[1].astype(jnp.int32)
    dst_rows = dst.reshape(IDX_ROWS, 128)
    src_rows = src.reshape(IDX_ROWS, 128)
    padg = (jnp.arange((IDX_PAD_G - IDX_ROWS) * 128, dtype=jnp.int32)
            % N_NODES).reshape(IDX_PAD_G - IDX_ROWS, 128)
    pad15 = jnp.zeros(((IDX_PAD - IDX_ROWS) // NC, 128), jnp.int32)
    half = IDX_ROWS // NC
    dst_pad = jnp.concatenate([dst_rows, padg])
    src_pad = jnp.concatenate([src_rows, padg])
    scat_idx = jnp.concatenate(
        [dst_rows[:half], pad15, dst_rows[half:], pad15])
    zeros = jnp.zeros((NPAD, D), jnp.float32)

    h, hW = tc_encode(x, atom_tables, W_input[:D])
    g0 = sc_gather(hW, dst_pad)
    msg = tc_msg0(g0, edge_attr, bond_tables, W_input[D:])

    inv_e = 1.0 / N_EDGES
    for i in range(N_LAYERS - 1):
        parts = sc_scatter(msg, scat_idx, zeros)
        nm = tc_sum_parts(parts)
        g = sc_gather(nm, src_pad)
        y, stats = tc_matmul_stats(g, msg, W_hidden)
        s = stats[0] * inv_e
        var = stats[1] * inv_e - s * s
        a = bn_gamma[i] / jnp.sqrt(var + EPS)
        b = bn_beta[i] - s * a
        msg = tc_affine_relu(y, a[None, :], b[None, :])

    parts = sc_scatter(msg, scat_idx, zeros)
    return tc_final(h, parts, W_output[:D], W_output[D:], b_output[None, :])


# speedup vs baseline: 1.0185x; 1.0185x over previous
import functools
import jax
import jax.numpy as jnp
from jax import lax
from jax.experimental import pallas as pl
from jax.experimental.pallas import tpu as pltpu
from jax.experimental.pallas import tpu_sc as plsc

D = 128
N_NODES = 10000
N_EDGES = 320000
EH = N_EDGES // 2
N_LAYERS = 5
EPS = 1e-5

NC = 2
NS = 16
NW = NC * NS
IDXH = EH // 128
CPWH = 40
IDXH_PAD = NW * CPWH
OPADH = IDXH_PAD * 128
RPC = IDXH // NC
CORE_PAD = NS * CPWH
NPAD = 10112
STRIPE = NPAD // NS


def _encode_body(x_ref, at_ref, wt_ref, h_ref, hw_ref):
    xb = x_ref[...]
    acc = jnp.zeros((x_ref.shape[0], D), jnp.float32)
    for f in range(at_ref.shape[0]):
        oh = (xb[:, f][:, None] == lax.broadcasted_iota(
            jnp.int32, (x_ref.shape[0], at_ref.shape[1]), 1)).astype(jnp.float32)
        acc = acc + jnp.dot(oh, at_ref[f], preferred_element_type=jnp.float32)
    h_ref[...] = acc
    hw_ref[...] = jnp.dot(acc, wt_ref[...], preferred_element_type=jnp.float32)


def tc_encode(x, atom_tables, w_top):
    Bn = 2000
    grid = (N_NODES // Bn,)
    return pl.pallas_call(
        _encode_body,
        grid=grid,
        in_specs=[
            pl.BlockSpec((Bn, x.shape[1]), lambda i: (i, 0)),
            pl.BlockSpec(atom_tables.shape, lambda i: (0, 0, 0)),
            pl.BlockSpec((D, D), lambda i: (0, 0)),
        ],
        out_specs=[
            pl.BlockSpec((Bn, D), lambda i: (i, 0)),
            pl.BlockSpec((Bn, D), lambda i: (i, 0)),
        ],
        out_shape=[
            jax.ShapeDtypeStruct((N_NODES, D), jnp.float32),
            jax.ShapeDtypeStruct((N_NODES, D), jnp.float32),
        ],
    )(x, atom_tables, w_top)


def _msg0_body(g_ref, ea_ref, bt_ref, wb_ref, o_ref):
    ea = ea_ref[...]
    acc = jnp.zeros((ea_ref.shape[0], D), jnp.float32)
    for f in range(bt_ref.shape[0]):
        oh = (ea[:, f][:, None] == lax.broadcasted_iota(
            jnp.int32, (ea_ref.shape[0], bt_ref.shape[1]), 1)).astype(jnp.float32)
        acc = acc + jnp.dot(oh, bt_ref[f], preferred_element_type=jnp.float32)
    o_ref[...] = g_ref[...] + jnp.dot(acc, wb_ref[...],
                                      preferred_element_type=jnp.float32)


def tc_msg0(g0, edge_attr, bond_tables, w_bot):
    Be = 4000
    grid = (EH // Be,)
    return pl.pallas_call(
        _msg0_body,
        grid=grid,
        in_specs=[
            pl.BlockSpec((Be, D), lambda i: (i, 0)),
            pl.BlockSpec((Be, edge_attr.shape[1]), lambda i: (i, 0)),
            pl.BlockSpec(bond_tables.shape, lambda i: (0, 0, 0)),
            pl.BlockSpec((D, D), lambda i: (0, 0)),
        ],
        out_specs=pl.BlockSpec((Be, D), lambda i: (i, 0)),
        out_shape=jax.ShapeDtypeStruct((EH, D), jnp.float32),
    )(g0, edge_attr, bond_tables, w_bot)


def _sum4_body(pa_ref, pb_ref, o_ref):
    o_ref[...] = (pa_ref[0] + pa_ref[1]) + (pb_ref[0] + pb_ref[1])


def tc_sum4(parts_a, parts_b):
    Bn = 2000
    grid = (N_NODES // Bn,)
    return pl.pallas_call(
        _sum4_body,
        grid=grid,
        in_specs=[
            pl.BlockSpec((2, Bn, D), lambda i: (0, i, 0)),
            pl.BlockSpec((2, Bn, D), lambda i: (0, i, 0)),
        ],
        out_specs=pl.BlockSpec((Bn, D), lambda i: (i, 0)),
        out_shape=jax.ShapeDtypeStruct((N_NODES, D), jnp.float32),
    )(parts_a, parts_b)


def _mm_stats_body(g_ref, m_ref, w_ref, y_ref, s_ref):
    g = g_ref[...]
    m = m_ref[...]
    up = jnp.concatenate([m[1:], m[:1]], axis=0)
    dn = jnp.concatenate([m[-1:], m[:-1]], axis=0)
    even = (lax.broadcasted_iota(jnp.int32, m.shape, 0) % 2) == 0
    u = g - jnp.where(even, up, dn)
    y = jnp.dot(u, w_ref[...], preferred_element_type=jnp.float32)
    y_ref[...] = y

    @pl.when(pl.program_id(0) == 0)
    def _():
        s_ref[...] = jnp.zeros_like(s_ref)

    s_ref[0, :] += jnp.sum(y, axis=0)
    s_ref[1, :] += jnp.sum(y * y, axis=0)


def tc_matmul_stats(g, msg, w_hidden):
    Be = 6400
    grid = (EH // Be,)
    return pl.pallas_call(
        _mm_stats_body,
        grid=grid,
        in_specs=[
            pl.BlockSpec((Be, D), lambda i: (i, 0)),
            pl.BlockSpec((Be, D), lambda i: (i, 0)),
            pl.BlockSpec((D, D), lambda i: (0, 0)),
        ],
        out_specs=[
            pl.BlockSpec((Be, D), lambda i: (i, 0)),
            pl.BlockSpec((8, D), lambda i: (0, 0)),
        ],
        out_shape=[
            jax.ShapeDtypeStruct((EH, D), jnp.float32),
            jax.ShapeDtypeStruct((8, D), jnp.float32),
        ],
    )(g, msg, w_hidden)


def _affine_relu_body(y_ref, a_ref, b_ref, o_ref):
    o_ref[...] = jnp.maximum(y_ref[...] * a_ref[...] + b_ref[...], 0.0)


def tc_affine_relu(y, a, b):
    Be = 8000
    grid = (EH // Be,)
    return pl.pallas_call(
        _affine_relu_body,
        grid=grid,
        in_specs=[
            pl.BlockSpec((Be, D), lambda i: (i, 0)),
            pl.BlockSpec((1, D), lambda i: (0, 0)),
            pl.BlockSpec((1, D), lambda i: (0, 0)),
        ],
        out_specs=pl.BlockSpec((Be, D), lambda i: (i, 0)),
        out_shape=jax.ShapeDtypeStruct((EH, D), jnp.float32),
    )(y, a, b)


def _final_body(h_ref, pa_ref, pb_ref, wt_ref, wb_ref, b_ref, o_ref):
    nm = (pa_ref[0] + pa_ref[1]) + (pb_ref[0] + pb_ref[1])
    r = (jnp.dot(h_ref[...], wt_ref[...], preferred_element_type=jnp.float32)
         + jnp.dot(nm, wb_ref[...], preferred_element_type=jnp.float32)
         + b_ref[...])
    o_ref[...] = jnp.maximum(r, 0.0)


def tc_final(h, parts_a, parts_b, w_top, w_bot, bias):
    Bn = 2000
    grid = (N_NODES // Bn,)
    return pl.pallas_call(
        _final_body,
        grid=grid,
        in_specs=[
            pl.BlockSpec((Bn, D), lambda i: (i, 0)),
            pl.BlockSpec((2, Bn, D), lambda i: (0, i, 0)),
            pl.BlockSpec((2, Bn, D), lambda i: (0, i, 0)),
            pl.BlockSpec((D, D), lambda i: (0, 0)),
            pl.BlockSpec((D, D), lambda i: (0, 0)),
            pl.BlockSpec((1, D), lambda i: (0, 0)),
        ],
        out_specs=pl.BlockSpec((Bn, D), lambda i: (i, 0)),
        out_shape=jax.ShapeDtypeStruct((N_NODES, D), jnp.float32),
    )(h, parts_a, parts_b, w_top, w_bot, bias)


def sc_gather(table, idx_rows):
    mesh = plsc.VectorSubcoreMesh(core_axis_name="c", subcore_axis_name="s")

    @functools.partial(
        pl.kernel, mesh=mesh,
        out_type=jax.ShapeDtypeStruct((OPADH, D), jnp.float32),
        scratch_types=[
            pltpu.VMEM((CPWH, 128), jnp.int32),
            pltpu.VMEM((4, 128, D), jnp.float32),
            pltpu.SemaphoreType.DMA((4,)),
            pltpu.SemaphoreType.DMA((4,)),
        ],
    )
    def k(table_hbm, idx_hbm, out_hbm, idx_v, rows_v, gsem, ssem):
        c = lax.axis_index("c")
        s = lax.axis_index("s")
        wid = s * NC + c
        lo = wid * CPWH
        pltpu.sync_copy(idx_hbm.at[pl.ds(lo, CPWH)], idx_v)

        def gath(t, b):
            @pl.when(t < CPWH)
            def _():
                pltpu.async_copy(table_hbm.at[idx_v.at[t]], rows_v.at[b],
                                 gsem.at[b])

        def put(t, b):
            pltpu.make_async_copy(table_hbm.at[idx_v.at[t]], rows_v.at[b],
                                  gsem.at[b]).wait()
            pltpu.async_copy(rows_v.at[b],
                             out_hbm.at[pl.ds((lo + t) * 128, 128)],
                             ssem.at[b])

        def drain(t, b):
            @pl.when(t >= 0)
            def _():
                pltpu.make_async_copy(rows_v.at[b],
                                      out_hbm.at[pl.ds((lo + t) * 128, 128)],
                                      ssem.at[b]).wait()

        gath(0, 0)
        gath(1, 1)

        @pl.loop(0, CPWH // 4)
        def _(i):
            for kk in range(4):
                t = i * 4 + kk
                put(t, kk)
                drain(t - 2, (kk + 2) % 4)
                gath(t + 2, (kk + 2) % 4)

        drain(CPWH - 2, (CPWH - 2) % 4)
        drain(CPWH - 1, (CPWH - 1) % 4)

    return k(table, idx_rows)


def sc_scatter(msg_h, idx_h, zeros):
    mesh = plsc.VectorSubcoreMesh(core_axis_name="c", subcore_axis_name="s")

    @functools.partial(
        pl.kernel, mesh=mesh,
        out_type=jax.ShapeDtypeStruct((NC, NPAD, D), jnp.float32),
        scratch_types=[
            pltpu.VMEM((CPWH, 128), jnp.int32),
            pltpu.VMEM((2, 128, D), jnp.float32),
            pltpu.VMEM_SHARED((NPAD, D), jnp.float32),
            pltpu.SemaphoreType.DMA((2,)),
        ],
    )
    def k(msg_hbm, idx_hbm, z_hbm, out_hbm, idx_v, rows_v, acc_sh, lsem):
        c = lax.axis_index("c")
        s = lax.axis_index("s")
        pltpu.sync_copy(z_hbm.at[pl.ds(s * STRIPE, STRIPE)],
                        acc_sh.at[pl.ds(s * STRIPE, STRIPE)])
        base_l = s * CPWH
        pltpu.sync_copy(idx_hbm.at[pl.ds(c * CORE_PAD + base_l, CPWH)], idx_v)
        plsc.subcore_barrier()

        def load(u, b):
            @pl.when((u < CPWH) & (base_l + u < RPC))
            def _():
                pltpu.async_copy(
                    msg_hbm.at[pl.ds((c * RPC + base_l + u) * 128, 128)],
                    rows_v.at[b], lsem.at[b])

        def add(u, b):
            @pl.when(base_l + u < RPC)
            def _():
                pltpu.make_async_copy(
                    msg_hbm.at[pl.ds((c * RPC + base_l + u) * 128, 128)],
                    rows_v.at[b], lsem.at[b]).wait()
                pltpu.sync_copy(rows_v.at[b], acc_sh.at[idx_v.at[u]], add=True)

        load(0, 0)
        load(1, 1)

        @pl.loop(0, CPWH // 2)
        def _(i):
            for kk in range(2):
                u = i * 2 + kk
                add(u, kk)
                load(u + 2, kk)

        plsc.subcore_barrier()
        pltpu.sync_copy(acc_sh.at[pl.ds(s * STRIPE, STRIPE)],
                        out_hbm.at[c, pl.ds(s * STRIPE, STRIPE)])

    return k(msg_h, idx_h, zeros)


def _pad_gather_idx(rows):
    pad = (jnp.arange((IDXH_PAD - IDXH) * 128, dtype=jnp.int32)
           % N_NODES).reshape(IDXH_PAD - IDXH, 128)
    return jnp.concatenate([rows, pad])


def _pad_scatter_idx(rows):
    pad = jnp.zeros((CORE_PAD - RPC, 128), jnp.int32)
    return jnp.concatenate([rows[:RPC], pad, rows[RPC:], pad])


def kernel(x, edge_index, edge_attr, atom_tables, bond_tables,
           W_input, W_hidden, W_output, b_output, bn_gamma, bn_beta):
    x = x.astype(jnp.int32)
    edge_attr = edge_attr.astype(jnp.int32)
    src = edge_index[0].astype(jnp.int32)
    dst = edge_index[1].astype(jnp.int32)
    dst_h = [dst[:EH].reshape(IDXH, 128), dst[EH:].reshape(IDXH, 128)]
    src_h = [src[:EH].reshape(IDXH, 128), src[EH:].reshape(IDXH, 128)]
    dst_g = [_pad_gather_idx(r) for r in dst_h]
    src_g = [_pad_gather_idx(r) for r in src_h]
    scat = [_pad_scatter_idx(r) for r in dst_h]
    ea_h = [edge_attr[:EH], edge_attr[EH:]]
    zeros = jnp.zeros((NPAD, D), jnp.float32)

    h, hW = tc_encode(x, atom_tables, W_input[:D])
    msg = [None, None]
    for j in range(2):
        g0 = sc_gather(hW, dst_g[j])
        msg[j] = tc_msg0(g0, ea_h[j], bond_tables, W_input[D:])

    inv_e = 1.0 / N_EDGES
    for i in range(N_LAYERS - 1):
        pa = sc_scatter(msg[0], scat[0], zeros)
        pb = sc_scatter(msg[1], scat[1], zeros)
        nm = tc_sum4(pa, pb)
        g = [sc_gather(nm, src_g[0]), sc_gather(nm, src_g[1])]
        ya, sta = tc_matmul_stats(g[0], msg[0], W_hidden)
        yb, stb = tc_matmul_stats(g[1], msg[1], W_hidden)
        s = (sta[0] + stb[0]) * inv_e
        var = (sta[1] + stb[1]) * inv_e - s * s
        a = bn_gamma[i] / jnp.sqrt(var + EPS)
        b = bn_beta[i] - s * a
        msg = [tc_affine_relu(ya, a[None, :], b[None, :]),
               tc_affine_relu(yb, a[None, :], b[None, :])]

    pa = sc_scatter(msg[0], scat[0], zeros)
    pb = sc_scatter(msg[1], scat[1], zeros)
    return tc_final(h, pa, pb, W_output[:D], W_output[D:], b_output[None, :])


# speedup vs baseline: 1.0234x; 1.0048x over previous
import functools
import jax
import jax.numpy as jnp
from jax import lax
from jax.experimental import pallas as pl
from jax.experimental.pallas import tpu as pltpu
from jax.experimental.pallas import tpu_sc as plsc

D = 128
N_NODES = 10000
N_EDGES = 320000
EH = N_EDGES // 2
N_LAYERS = 5
EPS = 1e-5

NC = 2
NS = 16
NW = NC * NS
IDXH = EH // 128
CPWH = 40
IDXH_PAD = NW * CPWH
OPADH = IDXH_PAD * 128
RPC = IDXH // NC
CORE_PAD = NS * CPWH
NPAD = 10112
STRIPE = NPAD // NS


def _encode_body(x_ref, at_ref, wt_ref, h_ref, hw_ref):
    xb = x_ref[...]
    acc = jnp.zeros((x_ref.shape[0], D), jnp.float32)
    for f in range(at_ref.shape[0]):
        oh = (xb[:, f][:, None] == lax.broadcasted_iota(
            jnp.int32, (x_ref.shape[0], at_ref.shape[1]), 1)).astype(jnp.float32)
        acc = acc + jnp.dot(oh, at_ref[f], preferred_element_type=jnp.float32)
    h_ref[...] = acc
    hw_ref[...] = jnp.dot(acc, wt_ref[...], preferred_element_type=jnp.float32)


def tc_encode(x, atom_tables, w_top):
    Bn = 2000
    grid = (N_NODES // Bn,)
    return pl.pallas_call(
        _encode_body,
        grid=grid,
        in_specs=[
            pl.BlockSpec((Bn, x.shape[1]), lambda i: (i, 0)),
            pl.BlockSpec(atom_tables.shape, lambda i: (0, 0, 0)),
            pl.BlockSpec((D, D), lambda i: (0, 0)),
        ],
        out_specs=[
            pl.BlockSpec((Bn, D), lambda i: (i, 0)),
            pl.BlockSpec((Bn, D), lambda i: (i, 0)),
        ],
        out_shape=[
            jax.ShapeDtypeStruct((N_NODES, D), jnp.float32),
            jax.ShapeDtypeStruct((N_NODES, D), jnp.float32),
        ],
    )(x, atom_tables, w_top)


def _msg0_body(g_ref, ea_ref, bt_ref, wb_ref, o_ref):
    ea = ea_ref[...]
    acc = jnp.zeros((ea_ref.shape[0], D), jnp.float32)
    for f in range(bt_ref.shape[0]):
        oh = (ea[:, f][:, None] == lax.broadcasted_iota(
            jnp.int32, (ea_ref.shape[0], bt_ref.shape[1]), 1)).astype(jnp.float32)
        acc = acc + jnp.dot(oh, bt_ref[f], preferred_element_type=jnp.float32)
    o_ref[...] = g_ref[...] + jnp.dot(acc, wb_ref[...],
                                      preferred_element_type=jnp.float32)


def tc_msg0(g0, edge_attr, bond_tables, w_bot):
    Be = 4000
    grid = (EH // Be,)
    return pl.pallas_call(
        _msg0_body,
        grid=grid,
        in_specs=[
            pl.BlockSpec((Be, D), lambda i: (i, 0)),
            pl.BlockSpec((Be, edge_attr.shape[1]), lambda i: (i, 0)),
            pl.BlockSpec(bond_tables.shape, lambda i: (0, 0, 0)),
            pl.BlockSpec((D, D), lambda i: (0, 0)),
        ],
        out_specs=pl.BlockSpec((Be, D), lambda i: (i, 0)),
        out_shape=jax.ShapeDtypeStruct((EH, D), jnp.float32),
    )(g0, edge_attr, bond_tables, w_bot)


def _sum4_body(pa_ref, pb_ref, o_ref):
    o_ref[...] = (pa_ref[0] + pa_ref[1]) + (pb_ref[0] + pb_ref[1])


def tc_sum4(parts_a, parts_b):
    Bn = 2000
    grid = (N_NODES // Bn,)
    return pl.pallas_call(
        _sum4_body,
        grid=grid,
        in_specs=[
            pl.BlockSpec((2, Bn, D), lambda i: (0, i, 0)),
            pl.BlockSpec((2, Bn, D), lambda i: (0, i, 0)),
        ],
        out_specs=pl.BlockSpec((Bn, D), lambda i: (i, 0)),
        out_shape=jax.ShapeDtypeStruct((N_NODES, D), jnp.float32),
    )(parts_a, parts_b)


def _mm_stats_body(g_ref, m_ref, w_ref, y_ref, s_ref):
    g = g_ref[...]
    m = m_ref[...]
    up = jnp.concatenate([m[1:], m[:1]], axis=0)
    dn = jnp.concatenate([m[-1:], m[:-1]], axis=0)
    even = (lax.broadcasted_iota(jnp.int32, m.shape, 0) % 2) == 0
    u = g - jnp.where(even, up, dn)
    y = jnp.dot(u, w_ref[...], preferred_element_type=jnp.float32)
    y_ref[...] = y

    @pl.when(pl.program_id(0) == 0)
    def _():
        s_ref[...] = jnp.zeros_like(s_ref)

    s_ref[0, :] += jnp.sum(y, axis=0)
    s_ref[1, :] += jnp.sum(y * y, axis=0)


def tc_matmul_stats(g, msg, w_hidden):
    Be = 8000
    grid = (EH // Be,)
    return pl.pallas_call(
        _mm_stats_body,
        grid=grid,
        in_specs=[
            pl.BlockSpec((Be, D), lambda i: (i, 0)),
            pl.BlockSpec((Be, D), lambda i: (i, 0)),
            pl.BlockSpec((D, D), lambda i: (0, 0)),
        ],
        out_specs=[
            pl.BlockSpec((Be, D), lambda i: (i, 0)),
            pl.BlockSpec((8, D), lambda i: (0, 0)),
        ],
        out_shape=[
            jax.ShapeDtypeStruct((EH, D), jnp.float32),
            jax.ShapeDtypeStruct((8, D), jnp.float32),
        ],
    )(g, msg, w_hidden)


def _affine_relu_body(y_ref, a_ref, b_ref, o_ref):
    o_ref[...] = jnp.maximum(y_ref[...] * a_ref[...] + b_ref[...], 0.0)


def tc_affine_relu(y, a, b):
    Be = 16000
    grid = (EH // Be,)
    return pl.pallas_call(
        _affine_relu_body,
        grid=grid,
        in_specs=[
            pl.BlockSpec((Be, D), lambda i: (i, 0)),
            pl.BlockSpec((1, D), lambda i: (0, 0)),
            pl.BlockSpec((1, D), lambda i: (0, 0)),
        ],
        out_specs=pl.BlockSpec((Be, D), lambda i: (i, 0)),
        out_shape=jax.ShapeDtypeStruct((EH, D), jnp.float32),
    )(y, a, b)


def _final_body(h_ref, pa_ref, pb_ref, wt_ref, wb_ref, b_ref, o_ref):
    nm = (pa_ref[0] + pa_ref[1]) + (pb_ref[0] + pb_ref[1])
    r = (jnp.dot(h_ref[...], wt_ref[...], preferred_element_type=jnp.float32)
         + jnp.dot(nm, wb_ref[...], preferred_element_type=jnp.float32)
         + b_ref[...])
    o_ref[...] = jnp.maximum(r, 0.0)


def tc_final(h, parts_a, parts_b, w_top, w_bot, bias):
    Bn = 2000
    grid = (N_NODES // Bn,)
    return pl.pallas_call(
        _final_body,
        grid=grid,
        in_specs=[
            pl.BlockSpec((Bn, D), lambda i: (i, 0)),
            pl.BlockSpec((2, Bn, D), lambda i: (0, i, 0)),
            pl.BlockSpec((2, Bn, D), lambda i: (0, i, 0)),
            pl.BlockSpec((D, D), lambda i: (0, 0)),
            pl.BlockSpec((D, D), lambda i: (0, 0)),
            pl.BlockSpec((1, D), lambda i: (0, 0)),
        ],
        out_specs=pl.BlockSpec((Bn, D), lambda i: (i, 0)),
        out_shape=jax.ShapeDtypeStruct((N_NODES, D), jnp.float32),
    )(h, parts_a, parts_b, w_top, w_bot, bias)


def sc_gather(table, idx_rows):
    mesh = plsc.VectorSubcoreMesh(core_axis_name="c", subcore_axis_name="s")

    @functools.partial(
        pl.kernel, mesh=mesh,
        out_type=jax.ShapeDtypeStruct((OPADH, D), jnp.float32),
        scratch_types=[
            pltpu.VMEM((CPWH, 128), jnp.int32),
            pltpu.VMEM((4, 128, D), jnp.float32),
            pltpu.SemaphoreType.DMA((4,)),
            pltpu.SemaphoreType.DMA((4,)),
        ],
    )
    def k(table_hbm, idx_hbm, out_hbm, idx_v, rows_v, gsem, ssem):
        c = lax.axis_index("c")
        s = lax.axis_index("s")
        wid = s * NC + c
        lo = wid * CPWH
        pltpu.sync_copy(idx_hbm.at[pl.ds(lo, CPWH)], idx_v)

        def gath(t, b):
            @pl.when(t < CPWH)
            def _():
                pltpu.async_copy(table_hbm.at[idx_v.at[t]], rows_v.at[b],
                                 gsem.at[b])

        def put(t, b):
            pltpu.make_async_copy(table_hbm.at[idx_v.at[t]], rows_v.at[b],
                                  gsem.at[b]).wait()
            pltpu.async_copy(rows_v.at[b],
                             out_hbm.at[pl.ds((lo + t) * 128, 128)],
                             ssem.at[b])

        def drain(t, b):
            @pl.when(t >= 0)
            def _():
                pltpu.make_async_copy(rows_v.at[b],
                                      out_hbm.at[pl.ds((lo + t) * 128, 128)],
                                      ssem.at[b]).wait()

        gath(0, 0)
        gath(1, 1)

        @pl.loop(0, CPWH // 4)
        def _(i):
            for kk in range(4):
                t = i * 4 + kk
                put(t, kk)
                drain(t - 2, (kk + 2) % 4)
                gath(t + 2, (kk + 2) % 4)

        drain(CPWH - 2, (CPWH - 2) % 4)
        drain(CPWH - 1, (CPWH - 1) % 4)

    return k(table, idx_rows)


def sc_scatter(msg_h, idx_h, zeros):
    mesh = plsc.VectorSubcoreMesh(core_axis_name="c", subcore_axis_name="s")

    @functools.partial(
        pl.kernel, mesh=mesh,
        out_type=jax.ShapeDtypeStruct((NC, NPAD, D), jnp.float32),
        scratch_types=[
            pltpu.VMEM((CPWH, 128), jnp.int32),
            pltpu.VMEM((2, 128, D), jnp.float32),
            pltpu.VMEM_SHARED((NPAD, D), jnp.float32),
            pltpu.SemaphoreType.DMA((2,)),
        ],
    )
    def k(msg_hbm, idx_hbm, z_hbm, out_hbm, idx_v, rows_v, acc_sh, lsem):
        c = lax.axis_index("c")
        s = lax.axis_index("s")
        pltpu.sync_copy(z_hbm.at[pl.ds(s * STRIPE, STRIPE)],
                        acc_sh.at[pl.ds(s * STRIPE, STRIPE)])
        base_l = s * CPWH
        pltpu.sync_copy(idx_hbm.at[pl.ds(c * CORE_PAD + base_l, CPWH)], idx_v)
        plsc.subcore_barrier()

        def load(u, b):
            @pl.when((u < CPWH) & (base_l + u < RPC))
            def _():
                pltpu.async_copy(
                    msg_hbm.at[pl.ds((c * RPC + base_l + u) * 128, 128)],
                    rows_v.at[b], lsem.at[b])

        def add(u, b):
            @pl.when(base_l + u < RPC)
            def _():
                pltpu.make_async_copy(
                    msg_hbm.at[pl.ds((c * RPC + base_l + u) * 128, 128)],
                    rows_v.at[b], lsem.at[b]).wait()
                pltpu.sync_copy(rows_v.at[b], acc_sh.at[idx_v.at[u]], add=True)

        load(0, 0)
        load(1, 1)

        @pl.loop(0, CPWH // 2)
        def _(i):
            for kk in range(2):
                u = i * 2 + kk
                add(u, kk)
                load(u + 2, kk)

        plsc.subcore_barrier()
        pltpu.sync_copy(acc_sh.at[pl.ds(s * STRIPE, STRIPE)],
                        out_hbm.at[c, pl.ds(s * STRIPE, STRIPE)])

    return k(msg_h, idx_h, zeros)


def _pad_gather_idx(rows):
    pad = (jnp.arange((IDXH_PAD - IDXH) * 128, dtype=jnp.int32)
           % N_NODES).reshape(IDXH_PAD - IDXH, 128)
    return jnp.concatenate([rows, pad])


def _pad_scatter_idx(rows):
    pad = jnp.zeros((CORE_PAD - RPC, 128), jnp.int32)
    return jnp.concatenate([rows[:RPC], pad, rows[RPC:], pad])


def kernel(x, edge_index, edge_attr, atom_tables, bond_tables,
           W_input, W_hidden, W_output, b_output, bn_gamma, bn_beta):
    x = x.astype(jnp.int32)
    edge_attr = edge_attr.astype(jnp.int32)
    src = edge_index[0].astype(jnp.int32)
    dst = edge_index[1].astype(jnp.int32)
    dst_h = [dst[:EH].reshape(IDXH, 128), dst[EH:].reshape(IDXH, 128)]
    src_h = [src[:EH].reshape(IDXH, 128), src[EH:].reshape(IDXH, 128)]
    dst_g = [_pad_gather_idx(r) for r in dst_h]
    src_g = [_pad_gather_idx(r) for r in src_h]
    scat = [_pad_scatter_idx(r) for r in dst_h]
    ea_h = [edge_attr[:EH], edge_attr[EH:]]
    zeros = jnp.zeros((NPAD, D), jnp.float32)

    h, hW = tc_encode(x, atom_tables, W_input[:D])
    msg = [None, None]
    for j in range(2):
        g0 = sc_gather(hW, dst_g[j])
        msg[j] = tc_msg0(g0, ea_h[j], bond_tables, W_input[D:])

    inv_e = 1.0 / N_EDGES
    for i in range(N_LAYERS - 1):
        pa = sc_scatter(msg[0], scat[0], zeros)
        pb = sc_scatter(msg[1], scat[1], zeros)
        nm = tc_sum4(pa, pb)
        g = [sc_gather(nm, src_g[0]), sc_gather(nm, src_g[1])]
        ya, sta = tc_matmul_stats(g[0], msg[0], W_hidden)
        yb, stb = tc_matmul_stats(g[1], msg[1], W_hidden)
        s = (sta[0] + stb[0]) * inv_e
        var = (sta[1] + stb[1]) * inv_e - s * s
        a = bn_gamma[i] / jnp.sqrt(var + EPS)
        b = bn_beta[i] - s * a
        msg = [tc_affine_relu(ya, a[None, :], b[None, :]),
               tc_affine_relu(yb, a[None, :], b[None, :])]

    pa = sc_scatter(msg[0], scat[0], zeros)
    pb = sc_scatter(msg[1], scat[1], zeros)
    return tc_final(h, pa, pb, W_output[:D], W_output[D:], b_output[None, :])
